# baseline (speedup 1.0000x reference)
import functools

import jax
import jax.numpy as jnp
from jax import lax
from jax.experimental import pallas as pl
from jax.experimental.pallas import tpu as pltpu, tpu_sc as plsc

_NUM_USERS = 25000
_N_NODES = 50000
_DIM = 32
_HALF = 16
_N_LAYERS = 3
_N_EDGES = 1_600_000

_NC = 2
_NS = 16

_EPT = _N_EDGES // _NS
_C = 400
_NCHUNK = _EPT // _C
_G = 5
_NSUP = _NCHUNK // _G
_NPAD = 51200
_NPT = _NPAD // _NS
_NPC = 200
_NPIECE = _NPT // _NPC


def _zero_fill(buf):
    @pl.loop(0, _NPC)
    def _z(j):
        buf[j, :] = jnp.zeros((_HALF,), jnp.float32)


def _lightgcn_body(uemb, iemb, ei, wgt, users, items,
                   tab_a, tab_b, idx2, w2, rows2, di, g):
    c = lax.axis_index("c")
    s = lax.axis_index("s")
    col = pl.multiple_of(c * _HALF, _HALF)
    node_base = s * _NPT
    chunk_base = s * _NCHUNK
    tab = tab_a
    acc = tab_b

    p0 = lambda: rows2.at[0, pl.ds(0, _NPC)]
    p1 = lambda: rows2.at[1, pl.ds(0, _NPC)]

    def idx_dma(sup_dyn, b):
        cb = chunk_base + sup_dyn * _G
        return (pltpu.make_async_copy(ei.at[:, pl.ds(cb, _G)],
                                      idx2.at[b], di.at[b]),
                pltpu.make_async_copy(wgt.at[pl.ds(cb, _G)],
                                      w2.at[b], di.at[b]))

    def idx_start(sup_dyn, b):
        d1, d2 = idx_dma(sup_dyn, b)
        d1.start()
        d2.start()

    def idx_wait(b):
        d1, d2 = idx_dma(0, b)
        d1.wait()
        d2.wait()

    def gather_dma(b, k, rb):
        return pltpu.make_async_copy(
            tab.at[idx2.at[b, 0, k]], rows2.at[rb], g.at[rb])

    def read_emb_piece(nb, vbuf):
        @pl.when(nb < _NUM_USERS)
        def _():
            pltpu.sync_copy(
                uemb.at[pl.ds(nb, _NPC), pl.ds(col, _HALF)], vbuf)

        @pl.when((nb >= _NUM_USERS) & (nb < _N_NODES))
        def _():
            pltpu.sync_copy(
                iemb.at[pl.ds(nb - _NUM_USERS, _NPC), pl.ds(col, _HALF)],
                vbuf)

    def rw_out_piece(nb, vbuf, write):
        @pl.when(nb < _NUM_USERS)
        def _():
            tgt = users.at[pl.ds(nb, _NPC), pl.ds(col, _HALF)]
            if write:
                pltpu.sync_copy(vbuf, tgt)
            else:
                pltpu.sync_copy(tgt, vbuf)

        @pl.when((nb >= _NUM_USERS) & (nb < _N_NODES))
        def _():
            tgt = items.at[pl.ds(nb - _NUM_USERS, _NPC), pl.ds(col, _HALF)]
            if write:
                pltpu.sync_copy(vbuf, tgt)
            else:
                pltpu.sync_copy(tgt, vbuf)

    for k in range(_NPIECE):
        nb = node_base + k * _NPC
        read_emb_piece(nb, p0())
        pltpu.sync_copy(p0(), tab_a.at[pl.ds(nb, _NPC)])
    _zero_fill(p1())
    for k in range(_NPIECE):
        nb = node_base + k * _NPC
        pltpu.sync_copy(p1(), tab_b.at[pl.ds(nb, _NPC)])
    plsc.subcore_barrier()

    for layer in range(_N_LAYERS):
        tab = tab_a if layer % 2 == 0 else tab_b
        acc = tab_b if layer % 2 == 0 else tab_a

        idx_start(0, 0)
        idx_wait(0)
        gather_dma(0, 0, 0).start()
        idx_start(1, 1)

        @pl.loop(0, _NSUP, step=2)
        def _pair(i0):
            for b in (0, 1):
                m = i0 + b
                for k in range(_G):
                    rp = (b + k) % 2
                    gather_dma(b, k, rp).wait()
                    if k == 3:
                        nxt = lax.rem(m + 1, _NSUP)
                        if b == 0:
                            @pl.when(i0 > 0)
                            def _():
                                idx_start(nxt, 1)
                        else:
                            idx_start(nxt, 0)
                    if k < _G - 1:
                        gather_dma(b, k + 1, 1 - rp).start()
                    else:
                        idx_wait(1 - b)
                        gather_dma(1 - b, 0, 1 - rp).start()

                    @pl.loop(0, _C // _HALF)
                    def _scale(grp):
                        wv = w2[b, k, pl.ds(grp * _HALF, _HALF)]
                        for j in range(_HALF):
                            e = grp * _HALF + j
                            rows2[rp, e, :] = rows2[rp, e, :] * wv[j]

                    pltpu.sync_copy(rows2.at[rp], acc.at[idx2.at[b, 1, k]],
                                    add=True)

        gather_dma(0, 0, 0).wait()
        plsc.subcore_barrier()

        last = layer == _N_LAYERS - 1
        for k in range(_NPIECE):
            nb = node_base + k * _NPC
            if layer == 0:
                pltpu.sync_copy(tab_a.at[pl.ds(nb, _NPC)], p0())
            else:
                rw_out_piece(nb, p0(), write=False)
            pltpu.sync_copy(acc.at[pl.ds(nb, _NPC)], p1())

            @pl.loop(0, _NPC)
            def _accum(j):
                ssum = rows2[0, j, :] + rows2[1, j, :]
                rows2[0, j, :] = ssum * 0.25 if last else ssum

            rw_out_piece(nb, p0(), write=True)
            if not last:
                _zero_fill(p0())
                pltpu.sync_copy(p0(), tab.at[pl.ds(nb, _NPC)])
        plsc.subcore_barrier()


@functools.partial(jax.jit, static_argnames=("interpret",))
def _lightgcn(uemb, iemb, ei, wgt, interpret=False):
    mesh = plsc.VectorSubcoreMesh(
        core_axis_name="c", subcore_axis_name="s",
        num_cores=_NC, num_subcores=_NS)
    return pl.kernel(
        _lightgcn_body,
        out_type=(jax.ShapeDtypeStruct((_NUM_USERS, _DIM), jnp.float32),
                  jax.ShapeDtypeStruct((_N_NODES - _NUM_USERS, _DIM),
                                       jnp.float32)),
        mesh=mesh,
        scratch_types=[
            pltpu.VMEM_SHARED((_NPAD, _HALF), jnp.float32),
            pltpu.VMEM_SHARED((_NPAD, _HALF), jnp.float32),
            pltpu.VMEM((2, 2, _G, _C), jnp.int32),
            pltpu.VMEM((2, _G, _C), jnp.float32),
            pltpu.VMEM((2, _C, _HALF), jnp.float32),
            pltpu.SemaphoreType.DMA((2,)),
            pltpu.SemaphoreType.DMA((2,)),
        ],
        compiler_params=pltpu.CompilerParams(use_tc_tiling_on_sc=False,
                                             needs_layout_passes=False),
        interpret=interpret,
    )(uemb, iemb, ei, wgt)


def kernel(user_emb, item_emb, edge_index, edge_weight, interpret=False):
    ei = edge_index.reshape(2, _NS * _NCHUNK, _C)
    wr = edge_weight.reshape(_NS * _NCHUNK, _C)
    return _lightgcn(user_emb, item_emb, ei, wr, interpret=interpret)


# speedup vs baseline: 1.1274x; 1.1274x over previous
import functools

import jax
import jax.numpy as jnp
from jax import lax
from jax.experimental import pallas as pl
from jax.experimental.pallas import tpu as pltpu, tpu_sc as plsc

_NUM_USERS = 25000
_N_NODES = 50000
_DIM = 32
_HALF = 16
_N_LAYERS = 3
_N_EDGES = 1_600_000

_NC = 2
_NS = 16

_EPT = _N_EDGES // _NS
_C = 400
_NCHUNK = _EPT // _C
_G = 5
_NSUP = _NCHUNK // _G
_NPAD = 51200
_NPT = _NPAD // _NS
_NPC = 200
_NPIECE = _NPT // _NPC


def _zero_fill(buf):
    @pl.loop(0, _NPC)
    def _z(j):
        buf[j, :] = jnp.zeros((_HALF,), jnp.float32)


def _lightgcn_body(uemb, iemb, ei, wgt, users, items,
                   tab_a, tab_b, idx2, w2, rows2, di, g, sc):
    c = lax.axis_index("c")
    s = lax.axis_index("s")
    col = pl.multiple_of(c * _HALF, _HALF)
    node_base = s * _NPT
    chunk_base = s * _NCHUNK
    tab = tab_a
    acc = tab_b

    p0 = lambda: rows2.at[0, pl.ds(0, _NPC)]
    p1 = lambda: rows2.at[1, pl.ds(0, _NPC)]

    def idx_dma(sup_dyn, b):
        cb = chunk_base + sup_dyn * _G
        return (pltpu.make_async_copy(ei.at[:, pl.ds(cb, _G)],
                                      idx2.at[b], di.at[b]),
                pltpu.make_async_copy(wgt.at[pl.ds(cb, _G)],
                                      w2.at[b], di.at[b]))

    def idx_start(sup_dyn, b):
        d1, d2 = idx_dma(sup_dyn, b)
        d1.start()
        d2.start()

    def idx_wait(b):
        d1, d2 = idx_dma(0, b)
        d1.wait()
        d2.wait()

    def gather_dma(b, k, rb):
        return pltpu.make_async_copy(
            tab.at[idx2.at[b, 0, k]], rows2.at[rb], g.at[rb])

    _HA, _HB = 192, _C - 192

    def scat_dma(b, k, rb, lo, n):
        return pltpu.make_async_copy(
            rows2.at[rb, pl.ds(lo, n)],
            acc.at[idx2.at[b, 1, k, pl.ds(lo, n)]], sc.at[rb])

    def scat_wait(rb):
        scat_dma(0, 0, rb, 0, _HA).wait()
        scat_dma(0, 0, rb, _HA, _HB).wait()

    def read_emb_piece(nb, vbuf):
        @pl.when(nb < _NUM_USERS)
        def _():
            pltpu.sync_copy(
                uemb.at[pl.ds(nb, _NPC), pl.ds(col, _HALF)], vbuf)

        @pl.when((nb >= _NUM_USERS) & (nb < _N_NODES))
        def _():
            pltpu.sync_copy(
                iemb.at[pl.ds(nb - _NUM_USERS, _NPC), pl.ds(col, _HALF)],
                vbuf)

    def rw_out_piece(nb, vbuf, write):
        @pl.when(nb < _NUM_USERS)
        def _():
            tgt = users.at[pl.ds(nb, _NPC), pl.ds(col, _HALF)]
            if write:
                pltpu.sync_copy(vbuf, tgt)
            else:
                pltpu.sync_copy(tgt, vbuf)

        @pl.when((nb >= _NUM_USERS) & (nb < _N_NODES))
        def _():
            tgt = items.at[pl.ds(nb - _NUM_USERS, _NPC), pl.ds(col, _HALF)]
            if write:
                pltpu.sync_copy(vbuf, tgt)
            else:
                pltpu.sync_copy(tgt, vbuf)

    for k in range(_NPIECE):
        nb = node_base + k * _NPC
        read_emb_piece(nb, p0())
        pltpu.sync_copy(p0(), tab_a.at[pl.ds(nb, _NPC)])
    _zero_fill(p1())
    for k in range(_NPIECE):
        nb = node_base + k * _NPC
        pltpu.sync_copy(p1(), tab_b.at[pl.ds(nb, _NPC)])
    plsc.subcore_barrier()

    for layer in range(_N_LAYERS):
        tab = tab_a if layer % 2 == 0 else tab_b
        acc = tab_b if layer % 2 == 0 else tab_a

        idx_start(0, 0)
        idx_wait(0)
        gather_dma(0, 0, 0).start()
        idx_start(1, 1)

        @pl.loop(0, _NSUP, step=2)
        def _pair(i0):
            for b in (0, 1):
                m = i0 + b
                for k in range(_G):
                    rp = (b + k) % 2
                    gather_dma(b, k, rp).wait()
                    if b == 0 and k == 0:
                        @pl.when(i0 > 0)
                        def _():
                            scat_wait(1 - rp)
                    else:
                        scat_wait(1 - rp)
                    if k == 3:
                        nxt = lax.rem(m + 1, _NSUP)
                        if b == 0:
                            @pl.when(i0 > 0)
                            def _():
                                idx_start(nxt, 1)
                        else:
                            idx_start(nxt, 0)
                    if k < _G - 1:
                        gather_dma(b, k + 1, 1 - rp).start()
                    else:
                        idx_wait(1 - b)
                        gather_dma(1 - b, 0, 1 - rp).start()

                    @pl.loop(0, _HA // _HALF)
                    def _scale_a(grp):
                        wv = w2[b, k, pl.ds(grp * _HALF, _HALF)]
                        for j in range(_HALF):
                            e = grp * _HALF + j
                            rows2[rp, e, :] = rows2[rp, e, :] * wv[j]

                    scat_dma(b, k, rp, 0, _HA).start()

                    @pl.loop(_HA // _HALF, _C // _HALF)
                    def _scale_b(grp):
                        wv = w2[b, k, pl.ds(grp * _HALF, _HALF)]
                        for j in range(_HALF):
                            e = grp * _HALF + j
                            rows2[rp, e, :] = rows2[rp, e, :] * wv[j]

                    scat_dma(b, k, rp, _HA, _HB).start()

        gather_dma(0, 0, 0).wait()
        scat_wait(1)
        plsc.subcore_barrier()

        last = layer == _N_LAYERS - 1
        for k in range(_NPIECE):
            nb = node_base + k * _NPC
            if layer == 0:
                pltpu.sync_copy(tab_a.at[pl.ds(nb, _NPC)], p0())
            else:
                rw_out_piece(nb, p0(), write=False)
            pltpu.sync_copy(acc.at[pl.ds(nb, _NPC)], p1())

            @pl.loop(0, _NPC)
            def _accum(j):
                ssum = rows2[0, j, :] + rows2[1, j, :]
                rows2[0, j, :] = ssum * 0.25 if last else ssum

            rw_out_piece(nb, p0(), write=True)
            if not last:
                _zero_fill(p0())
                pltpu.sync_copy(p0(), tab.at[pl.ds(nb, _NPC)])
        plsc.subcore_barrier()


@functools.partial(jax.jit, static_argnames=("interpret",))
def _lightgcn(uemb, iemb, ei, wgt, interpret=False):
    mesh = plsc.VectorSubcoreMesh(
        core_axis_name="c", subcore_axis_name="s",
        num_cores=_NC, num_subcores=_NS)
    return pl.kernel(
        _lightgcn_body,
        out_type=(jax.ShapeDtypeStruct((_NUM_USERS, _DIM), jnp.float32),
                  jax.ShapeDtypeStruct((_N_NODES - _NUM_USERS, _DIM),
                                       jnp.float32)),
        mesh=mesh,
        scratch_types=[
            pltpu.VMEM_SHARED((_NPAD, _HALF), jnp.float32),
            pltpu.VMEM_SHARED((_NPAD, _HALF), jnp.float32),
            pltpu.VMEM((2, 2, _G, _C), jnp.int32),
            pltpu.VMEM((2, _G, _C), jnp.float32),
            pltpu.VMEM((2, _C, _HALF), jnp.float32),
            pltpu.SemaphoreType.DMA((2,)),
            pltpu.SemaphoreType.DMA((2,)),
            pltpu.SemaphoreType.DMA((2,)),
        ],
        compiler_params=pltpu.CompilerParams(use_tc_tiling_on_sc=False,
                                             needs_layout_passes=False),
        interpret=interpret,
    )(uemb, iemb, ei, wgt)


def kernel(user_emb, item_emb, edge_index, edge_weight, interpret=False):
    ei = edge_index.reshape(2, _NS * _NCHUNK, _C)
    wr = edge_weight.reshape(_NS * _NCHUNK, _C)
    return _lightgcn(user_emb, item_emb, ei, wr, interpret=interpret)
